# baseline (speedup 1.0000x reference)
import functools

import jax
import jax.numpy as jnp
from jax import lax
from jax.experimental import pallas as pl
from jax.experimental.pallas import tpu as pltpu
from jax.experimental.pallas import tpu_sc as plsc

B = 16384
EMB = 64
HID = 512
EPS = 1e-5

NW = 32
CHUNK = 128
B_PER_W = B // NW
N_CHUNKS = B_PER_W // CHUNK

BM = 2048


def _sc_gather_body(u_hbm, v_hbm, embu_hbm, embv_hbm,
                    xu_hbm, xv_hbm,
                    uidx, vidx, urows, vrows, sem):
    wid = lax.axis_index("s") * 2 + lax.axis_index("c")
    pltpu.sync_copy(u_hbm.at[pl.ds(wid * N_CHUNKS, N_CHUNKS)], uidx)
    pltpu.sync_copy(v_hbm.at[pl.ds(wid * N_CHUNKS, N_CHUNKS)], vidx)
    copies = []
    for j in range(N_CHUNKS):
        copies.append(pltpu.make_async_copy(
            embu_hbm.at[uidx.at[j]], urows.at[pl.ds(j * CHUNK, CHUNK)], sem))
        copies.append(pltpu.make_async_copy(
            embv_hbm.at[vidx.at[j]], vrows.at[pl.ds(j * CHUNK, CHUNK)], sem))
    for cp in copies:
        cp.start()
    for cp in copies:
        cp.wait()
    base = wid * B_PER_W
    pltpu.sync_copy(urows, xu_hbm.at[pl.ds(base, B_PER_W)])
    pltpu.sync_copy(vrows, xv_hbm.at[pl.ds(base, B_PER_W)])


_sc_gather = functools.partial(
    pl.kernel,
    out_type=[jax.ShapeDtypeStruct((B, EMB), jnp.float32),
              jax.ShapeDtypeStruct((B, EMB), jnp.float32)],
    mesh=plsc.VectorSubcoreMesh(core_axis_name="c", subcore_axis_name="s"),
    scratch_types=[
        pltpu.VMEM((N_CHUNKS, CHUNK), jnp.int32),
        pltpu.VMEM((N_CHUNKS, CHUNK), jnp.int32),
        pltpu.VMEM((B_PER_W, EMB), jnp.float32),
        pltpu.VMEM((B_PER_W, EMB), jnp.float32),
        pltpu.SemaphoreType.DMA,
    ],
    compiler_params=pltpu.CompilerParams(use_tc_tiling_on_sc=False),
)(_sc_gather_body)


def _mlp_body(xu_ref, xv_ref, w1u_ref, w1v_ref,
              su_ref, bu_ref, sv_ref, bv_ref,
              s2_ref, b2_ref, w2_ref, c2_ref, out_ref):
    xu = jnp.maximum(xu_ref[...] * su_ref[...] + bu_ref[...], 0.0)
    xv = jnp.maximum(xv_ref[...] * sv_ref[...] + bv_ref[...], 0.0)
    h = (jnp.dot(xu, w1u_ref[...], preferred_element_type=jnp.float32)
         + jnp.dot(xv, w1v_ref[...], preferred_element_type=jnp.float32))
    h = jnp.maximum(h * s2_ref[...] + b2_ref[...], 0.0)
    out_ref[...] = (jnp.dot(h, w2_ref[...], preferred_element_type=jnp.float32)
                    + c2_ref[...])


def _mlp(xu, xv, w1u, w1v, su, bu, sv, bv, s2, b2, w2t, c2):
    grid = (B // BM,)
    return pl.pallas_call(
        _mlp_body,
        grid=grid,
        in_specs=[
            pl.BlockSpec((BM, EMB), lambda i: (i, 0)),
            pl.BlockSpec((BM, EMB), lambda i: (i, 0)),
            pl.BlockSpec((EMB, HID), lambda i: (0, 0)),
            pl.BlockSpec((EMB, HID), lambda i: (0, 0)),
            pl.BlockSpec((1, EMB), lambda i: (0, 0)),
            pl.BlockSpec((1, EMB), lambda i: (0, 0)),
            pl.BlockSpec((1, EMB), lambda i: (0, 0)),
            pl.BlockSpec((1, EMB), lambda i: (0, 0)),
            pl.BlockSpec((1, HID), lambda i: (0, 0)),
            pl.BlockSpec((1, HID), lambda i: (0, 0)),
            pl.BlockSpec((HID, 1), lambda i: (0, 0)),
            pl.BlockSpec((1, 1), lambda i: (0, 0)),
        ],
        out_specs=pl.BlockSpec((BM, 1), lambda i: (i, 0)),
        out_shape=jax.ShapeDtypeStruct((B, 1), jnp.float32),
    )(xu, xv, w1u, w1v, su, bu, sv, bv, s2, b2, w2t, c2)


def kernel(u, v, emb_u, emb_v, bn1_gamma, bn1_beta, W1, b1,
           bn2_gamma, bn2_beta, W2, b2):
    u2 = u.astype(jnp.int32).reshape(NW * N_CHUNKS, CHUNK)
    v2 = v.astype(jnp.int32).reshape(NW * N_CHUNKS, CHUNK)
    xu, xv = _sc_gather(u2, v2, emb_u, emb_v)

    inv = 1.0 / jnp.sqrt(jnp.float32(1.0 + EPS))
    scale1 = (bn1_gamma * inv).reshape(1, 2 * EMB)
    su, sv = scale1[:, :EMB], scale1[:, EMB:]
    bu = bn1_beta.reshape(1, 2 * EMB)[:, :EMB]
    bv = bn1_beta.reshape(1, 2 * EMB)[:, EMB:]
    s2 = (bn2_gamma * inv).reshape(1, HID)
    b2v = (b1 * inv * bn2_gamma + bn2_beta).reshape(1, HID)
    w1t = W1.T
    w1u, w1v = w1t[:EMB], w1t[EMB:]
    w2t = W2.T
    c2 = b2.reshape(1, 1)

    return _mlp(xu, xv, w1u, w1v, su, bu, sv, bv, s2, b2v, w2t, c2)


# speedup vs baseline: 1.5609x; 1.5609x over previous
import functools

import jax
import jax.numpy as jnp
from jax import lax
from jax.experimental import pallas as pl
from jax.experimental.pallas import tpu as pltpu
from jax.experimental.pallas import tpu_sc as plsc

B = 16384
EMB = 64
HID = 512
EPS = 1e-5

NW = 32
B_PER_W = B // NW
CHUNK = 32
N_CH = B_PER_W // CHUNK

BM = 2048


def _fire_rows(emb, ivm, rows, sem):
    lane = lax.iota(jnp.int32, 16)

    def fire(g, _):
        vec = ivm[pl.ds(g * 16, 16)]
        for j in range(16):
            idx = jnp.max(jnp.where(lane == j, vec, 0))
            pltpu.make_async_copy(emb.at[idx], rows.at[g * 16 + j], sem).start()
        return 0
    lax.fori_loop(0, B_PER_W // 16, fire, 0)


def _drain_rows(emb, rows, sem):
    def drain(i, _):
        pltpu.make_async_copy(emb.at[0], rows.at[0], sem).wait()
        return 0
    lax.fori_loop(0, B_PER_W, drain, 0, unroll=8)


def _sc_gather_body(u_hbm, v_hbm, emb_u, emb_v,
                    xu_hbm, xv_hbm,
                    ivm, rows, dsem):
    wid = lax.axis_index("s") * 2 + lax.axis_index("c")
    base = wid * B_PER_W
    pltpu.sync_copy(u_hbm.at[wid], ivm)
    _fire_rows(emb_u, ivm, rows, dsem)
    _drain_rows(emb_u, rows, dsem)
    pltpu.sync_copy(rows, xu_hbm.at[pl.ds(base, B_PER_W)])
    pltpu.sync_copy(v_hbm.at[wid], ivm)
    _fire_rows(emb_v, ivm, rows, dsem)
    _drain_rows(emb_v, rows, dsem)
    pltpu.sync_copy(rows, xv_hbm.at[pl.ds(base, B_PER_W)])


_sc_gather = functools.partial(
    pl.kernel,
    out_type=[jax.ShapeDtypeStruct((B, EMB), jnp.float32),
              jax.ShapeDtypeStruct((B, EMB), jnp.float32)],
    mesh=plsc.VectorSubcoreMesh(core_axis_name="c", subcore_axis_name="s"),
    scratch_types=[
        pltpu.VMEM((B_PER_W,), jnp.int32),
        pltpu.VMEM((B_PER_W, EMB), jnp.float32),
        pltpu.SemaphoreType.DMA,
    ],
    compiler_params=pltpu.CompilerParams(needs_layout_passes=False),
)(_sc_gather_body)


def _mlp_body(xu_ref, xv_ref, w1u_ref, w1v_ref,
              su_ref, bu_ref, sv_ref, bv_ref,
              s2_ref, b2_ref, w2_ref, c2_ref, out_ref):
    xu = jnp.maximum(xu_ref[...] * su_ref[...] + bu_ref[...], 0.0)
    xv = jnp.maximum(xv_ref[...] * sv_ref[...] + bv_ref[...], 0.0)
    h = (jnp.dot(xu, w1u_ref[...], preferred_element_type=jnp.float32)
         + jnp.dot(xv, w1v_ref[...], preferred_element_type=jnp.float32))
    h = jnp.maximum(h * s2_ref[...] + b2_ref[...], 0.0)
    out_ref[...] = (jnp.dot(h, w2_ref[...], preferred_element_type=jnp.float32)
                    + c2_ref[...])


def _mlp(xu, xv, w1u, w1v, su, bu, sv, bv, s2, b2, w2t, c2):
    grid = (B // BM,)
    return pl.pallas_call(
        _mlp_body,
        grid=grid,
        in_specs=[
            pl.BlockSpec((BM, EMB), lambda i: (i, 0)),
            pl.BlockSpec((BM, EMB), lambda i: (i, 0)),
            pl.BlockSpec((EMB, HID), lambda i: (0, 0)),
            pl.BlockSpec((EMB, HID), lambda i: (0, 0)),
            pl.BlockSpec((1, EMB), lambda i: (0, 0)),
            pl.BlockSpec((1, EMB), lambda i: (0, 0)),
            pl.BlockSpec((1, EMB), lambda i: (0, 0)),
            pl.BlockSpec((1, EMB), lambda i: (0, 0)),
            pl.BlockSpec((1, HID), lambda i: (0, 0)),
            pl.BlockSpec((1, HID), lambda i: (0, 0)),
            pl.BlockSpec((HID, 1), lambda i: (0, 0)),
            pl.BlockSpec((1, 1), lambda i: (0, 0)),
        ],
        out_specs=pl.BlockSpec((BM, 1), lambda i: (i, 0)),
        out_shape=jax.ShapeDtypeStruct((B, 1), jnp.float32),
    )(xu, xv, w1u, w1v, su, bu, sv, bv, s2, b2, w2t, c2)


def kernel(u, v, emb_u, emb_v, bn1_gamma, bn1_beta, W1, b1,
           bn2_gamma, bn2_beta, W2, b2):
    u2 = u.astype(jnp.int32).reshape(NW, B_PER_W)
    v2 = v.astype(jnp.int32).reshape(NW, B_PER_W)
    xu, xv = _sc_gather(u2, v2, emb_u, emb_v)

    inv = 1.0 / jnp.sqrt(jnp.float32(1.0 + EPS))
    scale1 = (bn1_gamma * inv).reshape(1, 2 * EMB)
    su, sv = scale1[:, :EMB], scale1[:, EMB:]
    bu = bn1_beta.reshape(1, 2 * EMB)[:, :EMB]
    bv = bn1_beta.reshape(1, 2 * EMB)[:, EMB:]
    s2 = (bn2_gamma * inv).reshape(1, HID)
    b2v = (b1 * inv * bn2_gamma + bn2_beta).reshape(1, HID)
    w1t = W1.T
    w1u, w1v = w1t[:EMB], w1t[EMB:]
    w2t = W2.T
    c2 = b2.reshape(1, 1)

    return _mlp(xu, xv, w1u, w1v, su, bu, sv, bv, s2, b2v, w2t, c2)


# speedup vs baseline: 1.5639x; 1.0019x over previous
import functools

import jax
import jax.numpy as jnp
from jax import lax
from jax.experimental import pallas as pl
from jax.experimental.pallas import tpu as pltpu
from jax.experimental.pallas import tpu_sc as plsc

B = 16384
EMB = 64
HID = 512
EPS = 1e-5

NW = 32
B_PER_W = B // NW
CHUNK = 32
N_CH = B_PER_W // CHUNK

BM = 2048


def _fire_rows(emb, ivm, rows, sem):
    lane = lax.iota(jnp.int32, 16)

    def fire(g, _):
        vec = ivm[pl.ds(g * 16, 16)]
        for j in range(16):
            idx = jnp.max(jnp.where(lane == j, vec, 0))
            pltpu.make_async_copy(emb.at[idx], rows.at[g * 16 + j], sem).start()
        return 0
    lax.fori_loop(0, B_PER_W // 16, fire, 0)


def _drain_rows(emb, rows, sem):
    def drain(i, _):
        pltpu.make_async_copy(emb.at[0], rows.at[0], sem).wait()
        return 0
    lax.fori_loop(0, B_PER_W, drain, 0, unroll=8)


def _sc_gather_body(u_hbm, v_hbm, emb_u, emb_v,
                    xu_hbm, xv_hbm,
                    ivm, rows, dsem):
    wid = lax.axis_index("s") * 2 + lax.axis_index("c")
    base = wid * B_PER_W
    pltpu.sync_copy(u_hbm.at[wid], ivm)
    _fire_rows(emb_u, ivm, rows, dsem)
    _drain_rows(emb_u, rows, dsem)
    pltpu.sync_copy(rows, xu_hbm.at[pl.ds(base, B_PER_W)])
    pltpu.sync_copy(v_hbm.at[wid], ivm)
    _fire_rows(emb_v, ivm, rows, dsem)
    _drain_rows(emb_v, rows, dsem)
    pltpu.sync_copy(rows, xv_hbm.at[pl.ds(base, B_PER_W)])


_sc_gather = functools.partial(
    pl.kernel,
    out_type=[jax.ShapeDtypeStruct((B, EMB), jnp.float32),
              jax.ShapeDtypeStruct((B, EMB), jnp.float32)],
    mesh=plsc.VectorSubcoreMesh(core_axis_name="c", subcore_axis_name="s"),
    scratch_types=[
        pltpu.VMEM((B_PER_W,), jnp.int32),
        pltpu.VMEM((B_PER_W, EMB), jnp.float32),
        pltpu.SemaphoreType.DMA,
    ],
    compiler_params=pltpu.CompilerParams(needs_layout_passes=False,
                                         use_tc_tiling_on_sc=True),
)(_sc_gather_body)


def _mlp_body(xu_ref, xv_ref, w1u_ref, w1v_ref,
              su_ref, bu_ref, sv_ref, bv_ref,
              s2_ref, b2_ref, w2_ref, c2_ref, out_ref):
    xu = jnp.maximum(xu_ref[...] * su_ref[...] + bu_ref[...], 0.0)
    xv = jnp.maximum(xv_ref[...] * sv_ref[...] + bv_ref[...], 0.0)
    h = (jnp.dot(xu, w1u_ref[...], preferred_element_type=jnp.float32)
         + jnp.dot(xv, w1v_ref[...], preferred_element_type=jnp.float32))
    h = jnp.maximum(h * s2_ref[...] + b2_ref[...], 0.0)
    out_ref[...] = (jnp.dot(h, w2_ref[...], preferred_element_type=jnp.float32)
                    + c2_ref[...])


def _mlp(xu, xv, w1u, w1v, su, bu, sv, bv, s2, b2, w2t, c2):
    grid = (B // BM,)
    return pl.pallas_call(
        _mlp_body,
        grid=grid,
        in_specs=[
            pl.BlockSpec((BM, EMB), lambda i: (i, 0)),
            pl.BlockSpec((BM, EMB), lambda i: (i, 0)),
            pl.BlockSpec((EMB, HID), lambda i: (0, 0)),
            pl.BlockSpec((EMB, HID), lambda i: (0, 0)),
            pl.BlockSpec((1, EMB), lambda i: (0, 0)),
            pl.BlockSpec((1, EMB), lambda i: (0, 0)),
            pl.BlockSpec((1, EMB), lambda i: (0, 0)),
            pl.BlockSpec((1, EMB), lambda i: (0, 0)),
            pl.BlockSpec((1, HID), lambda i: (0, 0)),
            pl.BlockSpec((1, HID), lambda i: (0, 0)),
            pl.BlockSpec((HID, 1), lambda i: (0, 0)),
            pl.BlockSpec((1, 1), lambda i: (0, 0)),
        ],
        out_specs=pl.BlockSpec((BM, 1), lambda i: (i, 0)),
        out_shape=jax.ShapeDtypeStruct((B, 1), jnp.float32),
    )(xu, xv, w1u, w1v, su, bu, sv, bv, s2, b2, w2t, c2)


def kernel(u, v, emb_u, emb_v, bn1_gamma, bn1_beta, W1, b1,
           bn2_gamma, bn2_beta, W2, b2):
    u2 = u.astype(jnp.int32).reshape(NW, B_PER_W)
    v2 = v.astype(jnp.int32).reshape(NW, B_PER_W)
    xu, xv = _sc_gather(u2, v2, emb_u, emb_v)

    inv = 1.0 / jnp.sqrt(jnp.float32(1.0 + EPS))
    scale1 = (bn1_gamma * inv).reshape(1, 2 * EMB)
    su, sv = scale1[:, :EMB], scale1[:, EMB:]
    bu = bn1_beta.reshape(1, 2 * EMB)[:, :EMB]
    bv = bn1_beta.reshape(1, 2 * EMB)[:, EMB:]
    s2 = (bn2_gamma * inv).reshape(1, HID)
    b2v = (b1 * inv * bn2_gamma + bn2_beta).reshape(1, HID)
    w1t = W1.T
    w1u, w1v = w1t[:EMB], w1t[EMB:]
    w2t = W2.T
    c2 = b2.reshape(1, 1)

    return _mlp(xu, xv, w1u, w1v, su, bu, sv, bv, s2, b2v, w2t, c2)


# speedup vs baseline: 1.5823x; 1.0118x over previous
import functools

import jax
import jax.numpy as jnp
from jax import lax
from jax.experimental import pallas as pl
from jax.experimental.pallas import tpu as pltpu
from jax.experimental.pallas import tpu_sc as plsc

B = 16384
EMB = 64
HID = 512
EPS = 1e-5

NW = 32
B_PER_W = B // NW
MAXB = B_PER_W
RING = 4

BM = 2048

_LANE = None


def _scal(ref, k):
    g = (k // 16) * 16
    vec = ref[pl.ds(g, 16)]
    lane = lax.iota(jnp.int32, 16)
    return jnp.max(jnp.where(lane == (k - g), vec, 0))


def _gather_one(embT, out_hbm, wid, mbv, jstv, jcntv, ev, pv, nbv,
                buf, rows, fsem, osem):
    nblk = _scal(nbv, 0)
    lane = lax.iota(jnp.int32, 16)

    def fetch(k):
        kc = jnp.minimum(k, nblk - 1)
        blk = _scal(mbv, kc)
        off = pl.multiple_of(blk * 128, 128)
        pltpu.make_async_copy(embT.at[:, pl.ds(off, 128)],
                              buf.at[k % RING], fsem.at[k % RING]).start()

    for k0 in range(RING - 1):
        fetch(jnp.int32(k0))

    def step(k, _):
        pltpu.make_async_copy(embT.at[:, pl.ds(0, 128)],
                              buf.at[0], fsem.at[k % RING]).wait()
        fetch(k + (RING - 1))
        jst = _scal(jstv, k)
        jcnt = _scal(jcntv, k)
        kmod16 = jnp.full((16,), k % RING, jnp.int32)

        def ext(j2, _):
            j = jst + j2
            e16 = jnp.full((16,), _scal(ev, j), jnp.int32)
            for m in range(EMB // 16):
                f16 = lane + (16 * m)
                val = plsc.load_gather(buf, [kmod16, f16, e16])
                rows[j, pl.ds(16 * m, 16)] = val
            p = _scal(pv, j)
            pltpu.make_async_copy(rows.at[j], out_hbm.at[p], osem).start()
            return 0

        lax.fori_loop(0, jcnt, ext, 0)
        return 0

    lax.fori_loop(0, nblk, step, 0)

    def drain_pref(i, _):
        pltpu.make_async_copy(embT.at[:, pl.ds(0, 128)],
                              buf.at[0], fsem.at[(nblk + i) % RING]).wait()
        return 0
    lax.fori_loop(0, RING - 1, drain_pref, 0)

    def drain(i, _):
        pltpu.make_async_copy(rows.at[0], out_hbm.at[0], osem).wait()
        return 0
    lax.fori_loop(0, B_PER_W, drain, 0, unroll=8)


def _sc_gather_body(mbu, jstu, jcntu, eu, pu, nbu,
                    mbv_h, jstv_h, jcntv_h, ev_h, pv_h, nbv_h,
                    embuT, embvT,
                    xu_hbm, xv_hbm,
                    mbv, jstv, jcntv, ev, pv, nbv, buf, rows, fsem, osem):
    wid = lax.axis_index("s") * 2 + lax.axis_index("c")

    def stage(mb_h, jst_h, jcnt_h, e_h, p_h, nb_h):
        pltpu.sync_copy(mb_h.at[wid], mbv)
        pltpu.sync_copy(jst_h.at[wid], jstv)
        pltpu.sync_copy(jcnt_h.at[wid], jcntv)
        pltpu.sync_copy(e_h.at[wid], ev)
        pltpu.sync_copy(p_h.at[wid], pv)
        pltpu.sync_copy(nb_h.at[wid], nbv)

    stage(mbu, jstu, jcntu, eu, pu, nbu)
    _gather_one(embuT, xu_hbm, wid, mbv, jstv, jcntv, ev, pv, nbv,
                buf, rows, fsem, osem)
    stage(mbv_h, jstv_h, jcntv_h, ev_h, pv_h, nbv_h)
    _gather_one(embvT, xv_hbm, wid, mbv, jstv, jcntv, ev, pv, nbv,
                buf, rows, fsem, osem)


_sc_gather = functools.partial(
    pl.kernel,
    out_type=[jax.ShapeDtypeStruct((B, EMB), jnp.float32),
              jax.ShapeDtypeStruct((B, EMB), jnp.float32)],
    mesh=plsc.VectorSubcoreMesh(core_axis_name="c", subcore_axis_name="s"),
    scratch_types=[
        pltpu.VMEM((MAXB,), jnp.int32),
        pltpu.VMEM((MAXB,), jnp.int32),
        pltpu.VMEM((MAXB,), jnp.int32),
        pltpu.VMEM((B_PER_W,), jnp.int32),
        pltpu.VMEM((B_PER_W,), jnp.int32),
        pltpu.VMEM((16,), jnp.int32),
        pltpu.VMEM((RING, EMB, 128), jnp.float32),
        pltpu.VMEM((B_PER_W, EMB), jnp.float32),
        pltpu.SemaphoreType.DMA((RING,)),
        pltpu.SemaphoreType.DMA,
    ],
    compiler_params=pltpu.CompilerParams(needs_layout_passes=False,
                                         use_tc_tiling_on_sc=True),
)(_sc_gather_body)


def _prep(idx32):
    iot = jnp.arange(B, dtype=jnp.int32)
    s, p = lax.sort_key_val(idx32, iot)
    blk = s >> 7
    e = s & 127
    first = jnp.concatenate(
        [jnp.ones((1,), jnp.int32), (blk[1:] != blk[:-1]).astype(jnp.int32)])
    slot = jnp.cumsum(first) - 1
    jst_g = jnp.full((B,), B, jnp.int32).at[slot].min(iot)
    jen_g = jnp.zeros((B,), jnp.int32).at[slot].max(iot)
    ub = jnp.zeros((B,), jnp.int32).at[slot].set(blk)
    w = jnp.arange(NW, dtype=jnp.int32)
    slo = slot[w * B_PER_W]
    shi = slot[w * B_PER_W + (B_PER_W - 1)]
    nblk = shi - slo + 1
    kar = jnp.arange(MAXB, dtype=jnp.int32)
    gslot = jnp.minimum(slo[:, None] + kar[None, :], B - 1)
    mb = ub[gslot]
    jst = jst_g[gslot]
    jen = jen_g[gslot] + 1
    lob = (w * B_PER_W)[:, None]
    hib = lob + B_PER_W
    jst_cl = jnp.clip(jst, lob, hib)
    jen_cl = jnp.clip(jen, lob, hib)
    jcnt = jnp.maximum(jen_cl - jst_cl, 0)
    jst_local = jst_cl - lob
    nb_arr = jnp.broadcast_to(nblk[:, None], (NW, 16)).astype(jnp.int32)
    e_arr = e.reshape(NW, B_PER_W)
    p_arr = p.reshape(NW, B_PER_W)
    return mb, jst_local, jcnt, e_arr, p_arr, nb_arr


def _mlp_body(xu_ref, xv_ref, w1u_ref, w1v_ref,
              su_ref, bu_ref, sv_ref, bv_ref,
              s2_ref, b2_ref, w2_ref, c2_ref, out_ref):
    xu = jnp.maximum(xu_ref[...] * su_ref[...] + bu_ref[...], 0.0)
    xv = jnp.maximum(xv_ref[...] * sv_ref[...] + bv_ref[...], 0.0)
    h = (jnp.dot(xu, w1u_ref[...], preferred_element_type=jnp.float32)
         + jnp.dot(xv, w1v_ref[...], preferred_element_type=jnp.float32))
    h = jnp.maximum(h * s2_ref[...] + b2_ref[...], 0.0)
    out_ref[...] = (jnp.dot(h, w2_ref[...], preferred_element_type=jnp.float32)
                    + c2_ref[...])


def _mlp(xu, xv, w1u, w1v, su, bu, sv, bv, s2, b2, w2t, c2):
    grid = (B // BM,)
    return pl.pallas_call(
        _mlp_body,
        grid=grid,
        in_specs=[
            pl.BlockSpec((BM, EMB), lambda i: (i, 0)),
            pl.BlockSpec((BM, EMB), lambda i: (i, 0)),
            pl.BlockSpec((EMB, HID), lambda i: (0, 0)),
            pl.BlockSpec((EMB, HID), lambda i: (0, 0)),
            pl.BlockSpec((1, EMB), lambda i: (0, 0)),
            pl.BlockSpec((1, EMB), lambda i: (0, 0)),
            pl.BlockSpec((1, EMB), lambda i: (0, 0)),
            pl.BlockSpec((1, EMB), lambda i: (0, 0)),
            pl.BlockSpec((1, HID), lambda i: (0, 0)),
            pl.BlockSpec((1, HID), lambda i: (0, 0)),
            pl.BlockSpec((HID, 1), lambda i: (0, 0)),
            pl.BlockSpec((1, 1), lambda i: (0, 0)),
        ],
        out_specs=pl.BlockSpec((BM, 1), lambda i: (i, 0)),
        out_shape=jax.ShapeDtypeStruct((B, 1), jnp.float32),
    )(xu, xv, w1u, w1v, su, bu, sv, bv, s2, b2, w2t, c2)


def kernel(u, v, emb_u, emb_v, bn1_gamma, bn1_beta, W1, b1,
           bn2_gamma, bn2_beta, W2, b2):
    u32 = u.astype(jnp.int32)
    v32 = v.astype(jnp.int32)
    pu = _prep(u32)
    pv = _prep(v32)
    xu, xv = _sc_gather(*pu, *pv, emb_u.T, emb_v.T)

    inv = 1.0 / jnp.sqrt(jnp.float32(1.0 + EPS))
    scale1 = (bn1_gamma * inv).reshape(1, 2 * EMB)
    su, sv = scale1[:, :EMB], scale1[:, EMB:]
    bu = bn1_beta.reshape(1, 2 * EMB)[:, :EMB]
    bv = bn1_beta.reshape(1, 2 * EMB)[:, EMB:]
    s2 = (bn2_gamma * inv).reshape(1, HID)
    b2v = (b1 * inv * bn2_gamma + bn2_beta).reshape(1, HID)
    w1t = W1.T
    w1u, w1v = w1t[:EMB], w1t[EMB:]
    w2t = W2.T
    c2 = b2.reshape(1, 1)

    return _mlp(xu, xv, w1u, w1v, su, bu, sv, bv, s2, b2v, w2t, c2)


# speedup vs baseline: 4.2633x; 2.6943x over previous
import functools

import jax
import jax.numpy as jnp
from jax import lax
from jax.experimental import pallas as pl
from jax.experimental.pallas import tpu as pltpu
from jax.experimental.pallas import tpu_sc as plsc

B = 16384
EMB = 64
HID = 512
EPS = 1e-5

NW = 32
B_PER_W = B // NW
MAXB = B_PER_W
RING = 4

BM = 2048

_LANE = None


def _scal(ref, k):
    g = (k // 16) * 16
    vec = ref[pl.ds(g, 16)]
    lane = lax.iota(jnp.int32, 16)
    return jnp.max(jnp.where(lane == (k - g), vec, 0))


def _gather_one(embT, out_hbm, wid, blkv, fstv, slv, ev, pv, nbv,
                mbv, jstv, buf, rows, fsem, osem):
    nblk = _scal(nbv, 0)
    lane = lax.iota(jnp.int32, 16)

    for g in range(MAXB // 16 + 1):
        jstv[pl.ds(g * 16, 16)] = jnp.full((16,), B_PER_W, jnp.int32)
    for g in range(B_PER_W // 16):
        sl16 = slv[pl.ds(g * 16, 16)]
        msk = fstv[pl.ds(g * 16, 16)] == 1
        plsc.store_scatter(mbv, [sl16], blkv[pl.ds(g * 16, 16)], mask=msk)
        plsc.store_scatter(jstv, [sl16], lane + (g * 16), mask=msk)

    def fetch(k):
        kc = jnp.minimum(k, nblk - 1)
        blk = _scal(mbv, kc)
        off = pl.multiple_of(blk * 128, 128)
        pltpu.make_async_copy(embT.at[:, pl.ds(off, 128)],
                              buf.at[k % RING], fsem.at[k % RING]).start()

    for k0 in range(RING - 1):
        fetch(jnp.int32(k0))

    def step(k, _):
        pltpu.make_async_copy(embT.at[:, pl.ds(0, 128)],
                              buf.at[0], fsem.at[k % RING]).wait()
        fetch(k + (RING - 1))
        jst = _scal(jstv, k)
        jcnt = _scal(jstv, k + 1) - jst
        kmod16 = jnp.full((16,), k % RING, jnp.int32)

        def ext(j2, _):
            j = jst + j2
            e16 = jnp.full((16,), _scal(ev, j), jnp.int32)
            for m in range(EMB // 16):
                f16 = lane + (16 * m)
                val = plsc.load_gather(buf, [kmod16, f16, e16])
                rows[j, pl.ds(16 * m, 16)] = val
            p = _scal(pv, j)
            pltpu.make_async_copy(rows.at[j], out_hbm.at[p], osem).start()
            return 0

        lax.fori_loop(0, jcnt, ext, 0)
        return 0

    lax.fori_loop(0, nblk, step, 0)

    def drain_pref(i, _):
        pltpu.make_async_copy(embT.at[:, pl.ds(0, 128)],
                              buf.at[0], fsem.at[(nblk + i) % RING]).wait()
        return 0
    lax.fori_loop(0, RING - 1, drain_pref, 0)

    def drain(i, _):
        pltpu.make_async_copy(rows.at[0], out_hbm.at[0], osem).wait()
        return 0
    lax.fori_loop(0, B_PER_W, drain, 0, unroll=8)


def _sc_gather_body(blku, fstu, slu, eu, pu, nbu,
                    blkv_h, fstv_h, slv_h, ev_h, pv_h, nbv_h,
                    embuT, embvT,
                    xu_hbm, xv_hbm,
                    blkv, fstv, slv, ev, pv, nbv,
                    mbv, jstv, buf, rows, fsem, osem):
    wid = lax.axis_index("s") * 2 + lax.axis_index("c")

    def stage(blk_h, fst_h, sl_h, e_h, p_h, nb_h):
        pltpu.sync_copy(blk_h.at[wid], blkv)
        pltpu.sync_copy(fst_h.at[wid], fstv)
        pltpu.sync_copy(sl_h.at[wid], slv)
        pltpu.sync_copy(e_h.at[wid], ev)
        pltpu.sync_copy(p_h.at[wid], pv)
        pltpu.sync_copy(nb_h.at[wid], nbv)

    stage(blku, fstu, slu, eu, pu, nbu)
    _gather_one(embuT, xu_hbm, wid, blkv, fstv, slv, ev, pv, nbv,
                mbv, jstv, buf, rows, fsem, osem)
    stage(blkv_h, fstv_h, slv_h, ev_h, pv_h, nbv_h)
    _gather_one(embvT, xv_hbm, wid, blkv, fstv, slv, ev, pv, nbv,
                mbv, jstv, buf, rows, fsem, osem)


_sc_gather = functools.partial(
    pl.kernel,
    out_type=[jax.ShapeDtypeStruct((B, EMB), jnp.float32),
              jax.ShapeDtypeStruct((B, EMB), jnp.float32)],
    mesh=plsc.VectorSubcoreMesh(core_axis_name="c", subcore_axis_name="s"),
    scratch_types=[
        pltpu.VMEM((B_PER_W,), jnp.int32),
        pltpu.VMEM((B_PER_W,), jnp.int32),
        pltpu.VMEM((B_PER_W,), jnp.int32),
        pltpu.VMEM((B_PER_W,), jnp.int32),
        pltpu.VMEM((B_PER_W,), jnp.int32),
        pltpu.VMEM((16,), jnp.int32),
        pltpu.VMEM((MAXB,), jnp.int32),
        pltpu.VMEM((MAXB + 16,), jnp.int32),
        pltpu.VMEM((RING, EMB, 128), jnp.float32),
        pltpu.VMEM((B_PER_W, EMB), jnp.float32),
        pltpu.SemaphoreType.DMA((RING,)),
        pltpu.SemaphoreType.DMA,
    ],
    compiler_params=pltpu.CompilerParams(needs_layout_passes=False,
                                         use_tc_tiling_on_sc=True),
)(_sc_gather_body)


def _prep(idx32):
    iot = jnp.arange(B, dtype=jnp.int32)
    s, p = lax.sort_key_val(idx32, iot)
    blk = s >> 7
    e = s & 127
    prev = jnp.concatenate([jnp.full((1,), -1, jnp.int32), blk[:-1]])
    bnd = (iot % B_PER_W) == 0
    first = ((blk != prev) | bnd).astype(jnp.int32)
    slot = jnp.cumsum(first) - 1
    slot2 = slot.reshape(NW, B_PER_W)
    slocal = slot2 - slot2[:, :1]
    nblk = slocal[:, -1:] + 1
    nb_arr = jnp.broadcast_to(nblk, (NW, 16)).astype(jnp.int32)
    return (blk.reshape(NW, B_PER_W), first.reshape(NW, B_PER_W), slocal,
            e.reshape(NW, B_PER_W), p.reshape(NW, B_PER_W), nb_arr)


def _mlp_body(xu_ref, xv_ref, w1u_ref, w1v_ref,
              su_ref, bu_ref, sv_ref, bv_ref,
              s2_ref, b2_ref, w2_ref, c2_ref, out_ref):
    xu = jnp.maximum(xu_ref[...] * su_ref[...] + bu_ref[...], 0.0)
    xv = jnp.maximum(xv_ref[...] * sv_ref[...] + bv_ref[...], 0.0)
    h = (jnp.dot(xu, w1u_ref[...], preferred_element_type=jnp.float32)
         + jnp.dot(xv, w1v_ref[...], preferred_element_type=jnp.float32))
    h = jnp.maximum(h * s2_ref[...] + b2_ref[...], 0.0)
    out_ref[...] = (jnp.dot(h, w2_ref[...], preferred_element_type=jnp.float32)
                    + c2_ref[...])


def _mlp(xu, xv, w1u, w1v, su, bu, sv, bv, s2, b2, w2t, c2):
    grid = (B // BM,)
    return pl.pallas_call(
        _mlp_body,
        grid=grid,
        in_specs=[
            pl.BlockSpec((BM, EMB), lambda i: (i, 0)),
            pl.BlockSpec((BM, EMB), lambda i: (i, 0)),
            pl.BlockSpec((EMB, HID), lambda i: (0, 0)),
            pl.BlockSpec((EMB, HID), lambda i: (0, 0)),
            pl.BlockSpec((1, EMB), lambda i: (0, 0)),
            pl.BlockSpec((1, EMB), lambda i: (0, 0)),
            pl.BlockSpec((1, EMB), lambda i: (0, 0)),
            pl.BlockSpec((1, EMB), lambda i: (0, 0)),
            pl.BlockSpec((1, HID), lambda i: (0, 0)),
            pl.BlockSpec((1, HID), lambda i: (0, 0)),
            pl.BlockSpec((HID, 1), lambda i: (0, 0)),
            pl.BlockSpec((1, 1), lambda i: (0, 0)),
        ],
        out_specs=pl.BlockSpec((BM, 1), lambda i: (i, 0)),
        out_shape=jax.ShapeDtypeStruct((B, 1), jnp.float32),
    )(xu, xv, w1u, w1v, su, bu, sv, bv, s2, b2, w2t, c2)


def kernel(u, v, emb_u, emb_v, bn1_gamma, bn1_beta, W1, b1,
           bn2_gamma, bn2_beta, W2, b2):
    u32 = u.astype(jnp.int32)
    v32 = v.astype(jnp.int32)
    pu = _prep(u32)
    pv = _prep(v32)
    xu, xv = _sc_gather(*pu, *pv, emb_u.T, emb_v.T)

    inv = 1.0 / jnp.sqrt(jnp.float32(1.0 + EPS))
    scale1 = (bn1_gamma * inv).reshape(1, 2 * EMB)
    su, sv = scale1[:, :EMB], scale1[:, EMB:]
    bu = bn1_beta.reshape(1, 2 * EMB)[:, :EMB]
    bv = bn1_beta.reshape(1, 2 * EMB)[:, EMB:]
    s2 = (bn2_gamma * inv).reshape(1, HID)
    b2v = (b1 * inv * bn2_gamma + bn2_beta).reshape(1, HID)
    w1t = W1.T
    w1u, w1v = w1t[:EMB], w1t[EMB:]
    w2t = W2.T
    c2 = b2.reshape(1, 1)

    return _mlp(xu, xv, w1u, w1v, su, bu, sv, bv, s2, b2v, w2t, c2)


# speedup vs baseline: 4.8449x; 1.1364x over previous
import functools

import jax
import jax.numpy as jnp
from jax import lax
from jax.experimental import pallas as pl
from jax.experimental.pallas import tpu as pltpu
from jax.experimental.pallas import tpu_sc as plsc

B = 16384
EMB = 64
HID = 512
EPS = 1e-5

NW = 32
B_PER_W = B // NW
MAXB = B_PER_W
RING = 6

BM = 2048

_LANE = None


def _scal(ref, k):
    g = (k // 16) * 16
    vec = ref[pl.ds(g, 16)]
    lane = lax.iota(jnp.int32, 16)
    return jnp.max(jnp.where(lane == (k - g), vec, 0))


def _gather_one(embT, out_hbm, wid, blkv, fstv, slv, ev, pv, nbv,
                mbv, jstv, buf, rows, fsem, osem):
    nblk = _scal(nbv, 0)
    lane = lax.iota(jnp.int32, 16)

    for g in range(MAXB // 16 + 1):
        jstv[pl.ds(g * 16, 16)] = jnp.full((16,), B_PER_W, jnp.int32)
    for g in range(B_PER_W // 16):
        sl16 = slv[pl.ds(g * 16, 16)]
        msk = fstv[pl.ds(g * 16, 16)] == 1
        plsc.store_scatter(mbv, [sl16], blkv[pl.ds(g * 16, 16)], mask=msk)
        plsc.store_scatter(jstv, [sl16], lane + (g * 16), mask=msk)

    def fetch(k):
        kc = jnp.minimum(k, nblk - 1)
        blk = _scal(mbv, kc)
        off = pl.multiple_of(blk * 128, 128)
        pltpu.make_async_copy(embT.at[:, pl.ds(off, 128)],
                              buf.at[k % RING], fsem.at[k % RING]).start()

    for k0 in range(RING - 1):
        fetch(jnp.int32(k0))

    def step(k, _):
        pltpu.make_async_copy(embT.at[:, pl.ds(0, 128)],
                              buf.at[0], fsem.at[k % RING]).wait()
        fetch(k + (RING - 1))
        jst = _scal(jstv, k)
        jcnt = _scal(jstv, k + 1) - jst
        kmod16 = jnp.full((16,), k % RING, jnp.int32)

        def ext(j2, _):
            j = jst + j2
            e16 = jnp.full((16,), _scal(ev, j), jnp.int32)
            for m in range(EMB // 16):
                f16 = lane + (16 * m)
                val = plsc.load_gather(buf, [kmod16, f16, e16])
                rows[j, pl.ds(16 * m, 16)] = val
            p = _scal(pv, j)
            pltpu.make_async_copy(rows.at[j], out_hbm.at[p], osem).start()
            return 0

        lax.fori_loop(0, jcnt, ext, 0)
        return 0

    lax.fori_loop(0, nblk, step, 0)

    def drain_pref(i, _):
        pltpu.make_async_copy(embT.at[:, pl.ds(0, 128)],
                              buf.at[0], fsem.at[(nblk + i) % RING]).wait()
        return 0
    lax.fori_loop(0, RING - 1, drain_pref, 0)

    def drain(i, _):
        pltpu.make_async_copy(rows.at[0], out_hbm.at[0], osem).wait()
        return 0
    lax.fori_loop(0, B_PER_W, drain, 0, unroll=8)


def _sc_gather_body(blku, fstu, slu, eu, pu, nbu,
                    blkv_h, fstv_h, slv_h, ev_h, pv_h, nbv_h,
                    embuT, embvT,
                    xu_hbm, xv_hbm,
                    blkv, fstv, slv, ev, pv, nbv,
                    mbv, jstv, buf, rows, fsem, osem):
    wid = lax.axis_index("s") * 2 + lax.axis_index("c")

    def stage(blk_h, fst_h, sl_h, e_h, p_h, nb_h):
        pltpu.sync_copy(blk_h.at[wid], blkv)
        pltpu.sync_copy(fst_h.at[wid], fstv)
        pltpu.sync_copy(sl_h.at[wid], slv)
        pltpu.sync_copy(e_h.at[wid], ev)
        pltpu.sync_copy(p_h.at[wid], pv)
        pltpu.sync_copy(nb_h.at[wid], nbv)

    stage(blku, fstu, slu, eu, pu, nbu)
    _gather_one(embuT, xu_hbm, wid, blkv, fstv, slv, ev, pv, nbv,
                mbv, jstv, buf, rows, fsem, osem)
    stage(blkv_h, fstv_h, slv_h, ev_h, pv_h, nbv_h)
    _gather_one(embvT, xv_hbm, wid, blkv, fstv, slv, ev, pv, nbv,
                mbv, jstv, buf, rows, fsem, osem)


_sc_gather = functools.partial(
    pl.kernel,
    out_type=[jax.ShapeDtypeStruct((B, EMB), jnp.float32),
              jax.ShapeDtypeStruct((B, EMB), jnp.float32)],
    mesh=plsc.VectorSubcoreMesh(core_axis_name="c", subcore_axis_name="s"),
    scratch_types=[
        pltpu.VMEM((B_PER_W,), jnp.int32),
        pltpu.VMEM((B_PER_W,), jnp.int32),
        pltpu.VMEM((B_PER_W,), jnp.int32),
        pltpu.VMEM((B_PER_W,), jnp.int32),
        pltpu.VMEM((B_PER_W,), jnp.int32),
        pltpu.VMEM((16,), jnp.int32),
        pltpu.VMEM((MAXB,), jnp.int32),
        pltpu.VMEM((MAXB + 16,), jnp.int32),
        pltpu.VMEM((RING, EMB, 128), jnp.float32),
        pltpu.VMEM((B_PER_W, EMB), jnp.float32),
        pltpu.SemaphoreType.DMA((RING,)),
        pltpu.SemaphoreType.DMA,
    ],
    compiler_params=pltpu.CompilerParams(needs_layout_passes=False,
                                         use_tc_tiling_on_sc=True),
)(_sc_gather_body)


def _prep(idx32):
    iot = jnp.arange(B, dtype=jnp.int32)
    s, p = lax.sort_key_val(idx32, iot)
    blk = s >> 7
    e = s & 127
    prev = jnp.concatenate([jnp.full((1,), -1, jnp.int32), blk[:-1]])
    bnd = (iot % B_PER_W) == 0
    first = ((blk != prev) | bnd).astype(jnp.int32)
    slot = jnp.cumsum(first) - 1
    slot2 = slot.reshape(NW, B_PER_W)
    slocal = slot2 - slot2[:, :1]
    nblk = slocal[:, -1:] + 1
    nb_arr = jnp.broadcast_to(nblk, (NW, 16)).astype(jnp.int32)
    return (blk.reshape(NW, B_PER_W), first.reshape(NW, B_PER_W), slocal,
            e.reshape(NW, B_PER_W), p.reshape(NW, B_PER_W), nb_arr)


def _mlp_body(xu_ref, xv_ref, w1u_ref, w1v_ref,
              su_ref, bu_ref, sv_ref, bv_ref,
              s2_ref, b2_ref, w2_ref, c2_ref, out_ref):
    xu = jnp.maximum(xu_ref[...] * su_ref[...] + bu_ref[...], 0.0)
    xv = jnp.maximum(xv_ref[...] * sv_ref[...] + bv_ref[...], 0.0)
    h = (jnp.dot(xu, w1u_ref[...], preferred_element_type=jnp.float32)
         + jnp.dot(xv, w1v_ref[...], preferred_element_type=jnp.float32))
    h = jnp.maximum(h * s2_ref[...] + b2_ref[...], 0.0)
    out_ref[...] = (jnp.dot(h, w2_ref[...], preferred_element_type=jnp.float32)
                    + c2_ref[...])


def _mlp(xu, xv, w1u, w1v, su, bu, sv, bv, s2, b2, w2t, c2):
    grid = (B // BM,)
    return pl.pallas_call(
        _mlp_body,
        grid=grid,
        in_specs=[
            pl.BlockSpec((BM, EMB), lambda i: (i, 0)),
            pl.BlockSpec((BM, EMB), lambda i: (i, 0)),
            pl.BlockSpec((EMB, HID), lambda i: (0, 0)),
            pl.BlockSpec((EMB, HID), lambda i: (0, 0)),
            pl.BlockSpec((1, EMB), lambda i: (0, 0)),
            pl.BlockSpec((1, EMB), lambda i: (0, 0)),
            pl.BlockSpec((1, EMB), lambda i: (0, 0)),
            pl.BlockSpec((1, EMB), lambda i: (0, 0)),
            pl.BlockSpec((1, HID), lambda i: (0, 0)),
            pl.BlockSpec((1, HID), lambda i: (0, 0)),
            pl.BlockSpec((HID, 1), lambda i: (0, 0)),
            pl.BlockSpec((1, 1), lambda i: (0, 0)),
        ],
        out_specs=pl.BlockSpec((BM, 1), lambda i: (i, 0)),
        out_shape=jax.ShapeDtypeStruct((B, 1), jnp.float32),
    )(xu, xv, w1u, w1v, su, bu, sv, bv, s2, b2, w2t, c2)


def kernel(u, v, emb_u, emb_v, bn1_gamma, bn1_beta, W1, b1,
           bn2_gamma, bn2_beta, W2, b2):
    u32 = u.astype(jnp.int32)
    v32 = v.astype(jnp.int32)
    pu = _prep(u32)
    pv = _prep(v32)
    xu, xv = _sc_gather(*pu, *pv, emb_u.T, emb_v.T)

    inv = 1.0 / jnp.sqrt(jnp.float32(1.0 + EPS))
    scale1 = (bn1_gamma * inv).reshape(1, 2 * EMB)
    su, sv = scale1[:, :EMB], scale1[:, EMB:]
    bu = bn1_beta.reshape(1, 2 * EMB)[:, :EMB]
    bv = bn1_beta.reshape(1, 2 * EMB)[:, EMB:]
    s2 = (bn2_gamma * inv).reshape(1, HID)
    b2v = (b1 * inv * bn2_gamma + bn2_beta).reshape(1, HID)
    w1t = W1.T
    w1u, w1v = w1t[:EMB], w1t[EMB:]
    w2t = W2.T
    c2 = b2.reshape(1, 1)

    return _mlp(xu, xv, w1u, w1v, su, bu, sv, bv, s2, b2v, w2t, c2)


# speedup vs baseline: 4.8786x; 1.0069x over previous
import functools

import jax
import jax.numpy as jnp
from jax import lax
from jax.experimental import pallas as pl
from jax.experimental.pallas import tpu as pltpu
from jax.experimental.pallas import tpu_sc as plsc

B = 16384
EMB = 64
HID = 512
EPS = 1e-5

NW = 32
B_PER_W = B // NW
MAXB = B_PER_W
RING = 7

BM = 2048

_LANE = None


def _scal(ref, k):
    g = (k // 16) * 16
    vec = ref[pl.ds(g, 16)]
    lane = lax.iota(jnp.int32, 16)
    return jnp.max(jnp.where(lane == (k - g), vec, 0))


def _gather_one(embT, out_hbm, wid, blkv, fstv, slv, ev, pv, nbv,
                mbv, jstv, buf, rows, fsem, osem):
    nblk = _scal(nbv, 0)
    lane = lax.iota(jnp.int32, 16)

    for g in range(MAXB // 16 + 1):
        jstv[pl.ds(g * 16, 16)] = jnp.full((16,), B_PER_W, jnp.int32)
    for g in range(B_PER_W // 16):
        sl16 = slv[pl.ds(g * 16, 16)]
        msk = fstv[pl.ds(g * 16, 16)] == 1
        plsc.store_scatter(mbv, [sl16], blkv[pl.ds(g * 16, 16)], mask=msk)
        plsc.store_scatter(jstv, [sl16], lane + (g * 16), mask=msk)

    def fetch(k):
        kc = jnp.minimum(k, nblk - 1)
        blk = _scal(mbv, kc)
        off = pl.multiple_of(blk * 128, 128)
        pltpu.make_async_copy(embT.at[:, pl.ds(off, 128)],
                              buf.at[k % RING], fsem.at[k % RING]).start()

    for k0 in range(RING - 1):
        fetch(jnp.int32(k0))

    def step(k, _):
        pltpu.make_async_copy(embT.at[:, pl.ds(0, 128)],
                              buf.at[0], fsem.at[k % RING]).wait()
        fetch(k + (RING - 1))
        jst = _scal(jstv, k)
        jcnt = _scal(jstv, k + 1) - jst
        kmod16 = jnp.full((16,), k % RING, jnp.int32)

        def ext(j2, _):
            j = jst + j2
            e16 = jnp.full((16,), _scal(ev, j), jnp.int32)
            for m in range(EMB // 16):
                f16 = lane + (16 * m)
                val = plsc.load_gather(buf, [kmod16, f16, e16])
                rows[j, pl.ds(16 * m, 16)] = val
            p = _scal(pv, j)
            pltpu.make_async_copy(rows.at[j], out_hbm.at[p], osem).start()
            return 0

        lax.fori_loop(0, jcnt, ext, 0)
        return 0

    lax.fori_loop(0, nblk, step, 0)

    def drain_pref(i, _):
        pltpu.make_async_copy(embT.at[:, pl.ds(0, 128)],
                              buf.at[0], fsem.at[(nblk + i) % RING]).wait()
        return 0
    lax.fori_loop(0, RING - 1, drain_pref, 0)

    def drain(i, _):
        pltpu.make_async_copy(rows.at[0], out_hbm.at[0], osem).wait()
        return 0
    lax.fori_loop(0, B_PER_W, drain, 0, unroll=8)


def _sc_gather_body(blku, fstu, slu, eu, pu, nbu,
                    blkv_h, fstv_h, slv_h, ev_h, pv_h, nbv_h,
                    embuT, embvT,
                    xu_hbm, xv_hbm,
                    blkv, fstv, slv, ev, pv, nbv,
                    mbv, jstv, buf, rows, fsem, osem):
    wid = lax.axis_index("s") * 2 + lax.axis_index("c")

    def stage(blk_h, fst_h, sl_h, e_h, p_h, nb_h):
        pltpu.sync_copy(blk_h.at[wid], blkv)
        pltpu.sync_copy(fst_h.at[wid], fstv)
        pltpu.sync_copy(sl_h.at[wid], slv)
        pltpu.sync_copy(e_h.at[wid], ev)
        pltpu.sync_copy(p_h.at[wid], pv)
        pltpu.sync_copy(nb_h.at[wid], nbv)

    stage(blku, fstu, slu, eu, pu, nbu)
    _gather_one(embuT, xu_hbm, wid, blkv, fstv, slv, ev, pv, nbv,
                mbv, jstv, buf, rows, fsem, osem)
    stage(blkv_h, fstv_h, slv_h, ev_h, pv_h, nbv_h)
    _gather_one(embvT, xv_hbm, wid, blkv, fstv, slv, ev, pv, nbv,
                mbv, jstv, buf, rows, fsem, osem)


_sc_gather = functools.partial(
    pl.kernel,
    out_type=[jax.ShapeDtypeStruct((B, EMB), jnp.float32),
              jax.ShapeDtypeStruct((B, EMB), jnp.float32)],
    mesh=plsc.VectorSubcoreMesh(core_axis_name="c", subcore_axis_name="s"),
    scratch_types=[
        pltpu.VMEM((B_PER_W,), jnp.int32),
        pltpu.VMEM((B_PER_W,), jnp.int32),
        pltpu.VMEM((B_PER_W,), jnp.int32),
        pltpu.VMEM((B_PER_W,), jnp.int32),
        pltpu.VMEM((B_PER_W,), jnp.int32),
        pltpu.VMEM((16,), jnp.int32),
        pltpu.VMEM((MAXB,), jnp.int32),
        pltpu.VMEM((MAXB + 16,), jnp.int32),
        pltpu.VMEM((RING, EMB, 128), jnp.float32),
        pltpu.VMEM((B_PER_W, EMB), jnp.float32),
        pltpu.SemaphoreType.DMA((RING,)),
        pltpu.SemaphoreType.DMA,
    ],
    compiler_params=pltpu.CompilerParams(needs_layout_passes=False,
                                         use_tc_tiling_on_sc=True),
)(_sc_gather_body)


def _prep(idx32):
    iot = jnp.arange(B, dtype=jnp.int32)
    s, p = lax.sort_key_val(idx32, iot)
    blk = s >> 7
    e = s & 127
    prev = jnp.concatenate([jnp.full((1,), -1, jnp.int32), blk[:-1]])
    bnd = (iot % B_PER_W) == 0
    first = ((blk != prev) | bnd).astype(jnp.int32)
    slot = jnp.cumsum(first) - 1
    slot2 = slot.reshape(NW, B_PER_W)
    slocal = slot2 - slot2[:, :1]
    nblk = slocal[:, -1:] + 1
    nb_arr = jnp.broadcast_to(nblk, (NW, 16)).astype(jnp.int32)
    return (blk.reshape(NW, B_PER_W), first.reshape(NW, B_PER_W), slocal,
            e.reshape(NW, B_PER_W), p.reshape(NW, B_PER_W), nb_arr)


def _mlp_body(xu_ref, xv_ref, w1u_ref, w1v_ref,
              su_ref, bu_ref, sv_ref, bv_ref,
              s2_ref, b2_ref, w2_ref, c2_ref, out_ref):
    xu = jnp.maximum(xu_ref[...] * su_ref[...] + bu_ref[...], 0.0)
    xv = jnp.maximum(xv_ref[...] * sv_ref[...] + bv_ref[...], 0.0)
    h = (jnp.dot(xu, w1u_ref[...], preferred_element_type=jnp.float32)
         + jnp.dot(xv, w1v_ref[...], preferred_element_type=jnp.float32))
    h = jnp.maximum(h * s2_ref[...] + b2_ref[...], 0.0)
    out_ref[...] = (jnp.dot(h, w2_ref[...], preferred_element_type=jnp.float32)
                    + c2_ref[...])


def _mlp(xu, xv, w1u, w1v, su, bu, sv, bv, s2, b2, w2t, c2):
    grid = (B // BM,)
    return pl.pallas_call(
        _mlp_body,
        grid=grid,
        in_specs=[
            pl.BlockSpec((BM, EMB), lambda i: (i, 0)),
            pl.BlockSpec((BM, EMB), lambda i: (i, 0)),
            pl.BlockSpec((EMB, HID), lambda i: (0, 0)),
            pl.BlockSpec((EMB, HID), lambda i: (0, 0)),
            pl.BlockSpec((1, EMB), lambda i: (0, 0)),
            pl.BlockSpec((1, EMB), lambda i: (0, 0)),
            pl.BlockSpec((1, EMB), lambda i: (0, 0)),
            pl.BlockSpec((1, EMB), lambda i: (0, 0)),
            pl.BlockSpec((1, HID), lambda i: (0, 0)),
            pl.BlockSpec((1, HID), lambda i: (0, 0)),
            pl.BlockSpec((HID, 1), lambda i: (0, 0)),
            pl.BlockSpec((1, 1), lambda i: (0, 0)),
        ],
        out_specs=pl.BlockSpec((BM, 1), lambda i: (i, 0)),
        out_shape=jax.ShapeDtypeStruct((B, 1), jnp.float32),
    )(xu, xv, w1u, w1v, su, bu, sv, bv, s2, b2, w2t, c2)


def kernel(u, v, emb_u, emb_v, bn1_gamma, bn1_beta, W1, b1,
           bn2_gamma, bn2_beta, W2, b2):
    u32 = u.astype(jnp.int32)
    v32 = v.astype(jnp.int32)
    pu = _prep(u32)
    pv = _prep(v32)
    xu, xv = _sc_gather(*pu, *pv, emb_u.T, emb_v.T)

    inv = 1.0 / jnp.sqrt(jnp.float32(1.0 + EPS))
    scale1 = (bn1_gamma * inv).reshape(1, 2 * EMB)
    su, sv = scale1[:, :EMB], scale1[:, EMB:]
    bu = bn1_beta.reshape(1, 2 * EMB)[:, :EMB]
    bv = bn1_beta.reshape(1, 2 * EMB)[:, EMB:]
    s2 = (bn2_gamma * inv).reshape(1, HID)
    b2v = (b1 * inv * bn2_gamma + bn2_beta).reshape(1, HID)
    w1t = W1.T
    w1u, w1v = w1t[:EMB], w1t[EMB:]
    w2t = W2.T
    c2 = b2.reshape(1, 1)

    return _mlp(xu, xv, w1u, w1v, su, bu, sv, bv, s2, b2v, w2t, c2)
